# baseline (speedup 1.0000x reference)
import jax
import jax.numpy as jnp
from jax.experimental import pallas as pl
from jax.experimental.pallas import tpu as pltpu

_NUM_EMB = 1024
_DIM = 64
_BETA = 0.25
_ROWS = 16384
_BLOCK = 1024
_GRID = _ROWS // _BLOCK


def _vq_body(z_ref, w_ref, oh_ref, idx_ref, zq_ref, loss_ref, w2c_ref,
             codes_ref, wm2_ref):
    w = w_ref[...]

    @pl.when(pl.program_id(0) == 0)
    def _init():
        w2 = jnp.sum(w * w, axis=1)
        w2c_ref[...] = jnp.broadcast_to(w2[:, None], (_NUM_EMB, 128))
        codes_ref[...] = jax.lax.broadcasted_iota(
            jnp.int32, (_NUM_EMB, 128), 0).astype(jnp.float32)
        wm2_ref[...] = -2.0 * w
        loss_ref[...] = jnp.zeros((1, 1), jnp.float32)

    z_c = z_ref[0]
    z2 = jnp.sum(z_c * z_c, axis=0)
    prod_m2 = jax.lax.dot_general(
        wm2_ref[...], z_c, (((1,), (0,)), ((), ())),
        preferred_element_type=jnp.float32)
    _H = _NUM_EMB // 2
    obj_a = (z2[None, :] + w2c_ref[0:_H, 0:1]) + prod_m2[0:_H, :]
    obj_b = (z2[None, :] + w2c_ref[_H:, 0:1]) + prod_m2[_H:, :]
    m = jnp.minimum(jnp.min(obj_a, axis=0), jnp.min(obj_b, axis=0))
    big = jnp.float32(_NUM_EMB)
    idx_f = jnp.minimum(
        jnp.min(jnp.where(obj_a == m[None, :], codes_ref[0:_H, 0:1], big),
                axis=0),
        jnp.min(jnp.where(obj_b == m[None, :], codes_ref[_H:, 0:1], big),
                axis=0))
    idx = idx_f.astype(jnp.int32)

    codes_r = jax.lax.broadcasted_iota(jnp.int32, (_BLOCK, _NUM_EMB), 1)
    oh = jnp.where(codes_r == idx[:, None], 1.0, 0.0)
    oh_ref[...] = oh
    idx_ref[...] = idx

    zq_c = jax.lax.dot_general(
        w, oh, (((0,), (1,)), ((), ())),
        preferred_element_type=jnp.float32)
    zq_ref[...] = zq_c[None]

    loss_ref[...] += jnp.sum(m).reshape(1, 1)


def kernel(z, emb_w):
    z3 = z.reshape(16, 64, 1024)
    oh, idx, zq3, loss_sum = pl.pallas_call(
        _vq_body,
        grid=(_GRID,),
        in_specs=[
            pl.BlockSpec((1, _DIM, _BLOCK), lambda g: (g, 0, 0)),
            pl.BlockSpec((_NUM_EMB, _DIM), lambda g: (0, 0)),
        ],
        out_specs=[
            pl.BlockSpec((_BLOCK, _NUM_EMB), lambda g: (g, 0)),
            pl.BlockSpec((_BLOCK,), lambda g: (g,)),
            pl.BlockSpec((1, _DIM, _BLOCK), lambda g: (g, 0, 0)),
            pl.BlockSpec((1, 1), lambda g: (0, 0)),
        ],
        out_shape=[
            jax.ShapeDtypeStruct((_ROWS, _NUM_EMB), jnp.float32),
            jax.ShapeDtypeStruct((_ROWS,), jnp.int32),
            jax.ShapeDtypeStruct((16, _DIM, 1024), jnp.float32),
            jax.ShapeDtypeStruct((1, 1), jnp.float32),
        ],
        scratch_shapes=[pltpu.VMEM((_NUM_EMB, 128), jnp.float32),
                        pltpu.VMEM((_NUM_EMB, 128), jnp.float32),
                        pltpu.VMEM((_NUM_EMB, _DIM), jnp.float32)],
    )(z3, emb_w)
    loss = (1.0 + _BETA) * loss_sum[0, 0] / (_ROWS * _DIM)
    z_quantized = zq3.reshape(16, 64, 32, 32)
    return (loss, z_quantized, oh, idx)


# speedup vs baseline: 1.0441x; 1.0441x over previous
import functools as _ft

import jax
import jax.numpy as jnp
from jax.experimental import pallas as pl
from jax.experimental.pallas import tpu as pltpu

_NUM_EMB = 1024
_DIM = 64
_BETA = 0.25
_ROWS = 16384
_BLOCK = 1024
_GRID = _ROWS // _BLOCK


def _vq_body(z_ref, w_ref, oh_ref, idx_ref, zq_ref, loss_ref, w2c_ref,
             codes_ref, wm2_ref):
    w = w_ref[...]

    @pl.when(pl.program_id(0) == 0)
    def _init():
        w2 = jnp.sum(w * w, axis=1)
        w2c_ref[...] = jnp.broadcast_to(w2[:, None], (_NUM_EMB, 128))
        codes_ref[...] = jax.lax.broadcasted_iota(
            jnp.int32, (_NUM_EMB, 128), 0).astype(jnp.float32)
        wm2_ref[...] = -2.0 * w
        loss_ref[...] = jnp.zeros((1, 1), jnp.float32)

    z_c = z_ref[0]
    z2 = jnp.sum(z_c * z_c, axis=0)
    prod_m2 = jax.lax.dot_general(
        wm2_ref[...], z_c, (((1,), (0,)), ((), ())),
        preferred_element_type=jnp.float32)
    _H = _NUM_EMB // 4
    objs = [(z2[None, :] + w2c_ref[i * _H:(i + 1) * _H, 0:1])
            + prod_m2[i * _H:(i + 1) * _H, :] for i in range(4)]
    m = _ft.reduce(jnp.minimum, [jnp.min(o, axis=0) for o in objs])
    big = jnp.float32(_NUM_EMB)
    idx_f = _ft.reduce(jnp.minimum, [
        jnp.min(jnp.where(o == m[None, :],
                          codes_ref[i * _H:(i + 1) * _H, 0:1], big), axis=0)
        for i, o in enumerate(objs)])
    idx = idx_f.astype(jnp.int32)

    codes_r = jax.lax.broadcasted_iota(jnp.int32, (_BLOCK, _NUM_EMB), 1)
    oh = jnp.where(codes_r == idx[:, None], 1.0, 0.0)
    oh_ref[...] = oh
    idx_ref[...] = idx

    zq_c = jax.lax.dot_general(
        w, oh, (((0,), (1,)), ((), ())),
        preferred_element_type=jnp.float32)
    zq_ref[...] = zq_c[None]

    loss_ref[...] += jnp.sum(m).reshape(1, 1)


def kernel(z, emb_w):
    z3 = z.reshape(16, 64, 1024)
    oh, idx, zq3, loss_sum = pl.pallas_call(
        _vq_body,
        grid=(_GRID,),
        in_specs=[
            pl.BlockSpec((1, _DIM, _BLOCK), lambda g: (g, 0, 0)),
            pl.BlockSpec((_NUM_EMB, _DIM), lambda g: (0, 0)),
        ],
        out_specs=[
            pl.BlockSpec((_BLOCK, _NUM_EMB), lambda g: (g, 0)),
            pl.BlockSpec((_BLOCK,), lambda g: (g,)),
            pl.BlockSpec((1, _DIM, _BLOCK), lambda g: (g, 0, 0)),
            pl.BlockSpec((1, 1), lambda g: (0, 0)),
        ],
        out_shape=[
            jax.ShapeDtypeStruct((_ROWS, _NUM_EMB), jnp.float32),
            jax.ShapeDtypeStruct((_ROWS,), jnp.int32),
            jax.ShapeDtypeStruct((16, _DIM, 1024), jnp.float32),
            jax.ShapeDtypeStruct((1, 1), jnp.float32),
        ],
        scratch_shapes=[pltpu.VMEM((_NUM_EMB, 128), jnp.float32),
                        pltpu.VMEM((_NUM_EMB, 128), jnp.float32),
                        pltpu.VMEM((_NUM_EMB, _DIM), jnp.float32)],
    )(z3, emb_w)
    loss = (1.0 + _BETA) * loss_sum[0, 0] / (_ROWS * _DIM)
    z_quantized = zq3.reshape(16, 64, 32, 32)
    return (loss, z_quantized, oh, idx)
